# baseline (speedup 1.0000x reference)
import functools

import jax
import jax.numpy as jnp
from jax import lax
from jax.experimental import pallas as pl
from jax.experimental.pallas import tpu as pltpu
from jax.experimental.pallas import tpu_sc as plsc

_NC = 2
_NS = 16
_NW = _NC * _NS


def _mesh():
    return plsc.VectorSubcoreMesh(
        core_axis_name="c", subcore_axis_name="s", num_cores=_NC,
        num_subcores=_NS)


def _make_gather(n_rows, d, b, c_pad, chunk, nbuf):
    b_per_w = b // _NW
    nch = b_per_w // chunk

    @functools.partial(
        pl.kernel,
        mesh=_mesh(),
        out_type=(
            jax.ShapeDtypeStruct((b, d), jnp.float32),
            jax.ShapeDtypeStruct((b, c_pad), jnp.float32),
        ),
        scratch_types=[
            pltpu.VMEM((b_per_w,), jnp.int32),
            pltpu.VMEM((nbuf, chunk, d), jnp.float32),
            pltpu.VMEM((b_per_w, c_pad), jnp.float32),
        ]
        + [pltpu.SemaphoreType.DMA] * (2 * nbuf + 1),
    )
    def gather_kernel(spec_hbm, coords_hbm, idx_hbm, out_hbm, lab_hbm,
                      idx_v, rows_v, crows_v, *sems):
        gsems = sems[:nbuf]
        ssems = sems[nbuf:2 * nbuf]
        csem = sems[2 * nbuf]
        wid = lax.axis_index("s") * _NC + lax.axis_index("c")
        base = wid * b_per_w

        pltpu.sync_copy(idx_hbm.at[pl.ds(base, b_per_w)], idx_v)

        cop = pltpu.async_copy(coords_hbm.at[idx_v], crows_v, csem)

        def start_gather(c):
            buf = c % nbuf
            return pltpu.async_copy(
                spec_hbm.at[idx_v.at[pl.ds(c * chunk, chunk)]],
                rows_v.at[buf], gsems[buf])

        gathers = [None] * nbuf
        for c in range(min(nbuf, nch)):
            gathers[c] = start_gather(c)
        scatters = [None] * nbuf
        for c in range(nch):
            buf = c % nbuf
            gathers[buf].wait()
            scatters[buf] = pltpu.async_copy(
                rows_v.at[buf],
                out_hbm.at[pl.ds(base + c * chunk, chunk)], ssems[buf])
            if c + nbuf < nch:
                scatters[buf].wait()
                gathers[buf] = start_gather(c + nbuf)
        for c in range(max(0, nch - nbuf), nch):
            if scatters[c % nbuf] is not None:
                scatters[c % nbuf].wait()

        cop.wait()
        pltpu.sync_copy(crows_v, lab_hbm.at[pl.ds(base, b_per_w)])

    return gather_kernel


def kernel(spectrograms, coords, indices):
    n, h, w = spectrograms.shape
    d = h * w
    b = indices.shape[0]
    c_dim = coords.shape[1]
    c_pad = 128
    coords_p = jnp.pad(coords, ((0, 0), (0, c_pad - c_dim)))
    spec2d = spectrograms.reshape(n, d)
    samples, labels = _make_gather(n, d, b, c_pad, 8, 3)(
        spec2d, coords_p, indices)
    return samples.reshape(b, 1, h, w), labels[:, :c_dim]
